# Initial kernel scaffold; baseline (speedup 1.0000x reference)
import jax
import jax.numpy as jnp
from jax.experimental import pallas as pl


def kernel(input, pos, token_table, pos_table, gamma, beta, W, b):
    raise NotImplementedError("write your pallas kernel here")


# speedup vs baseline: 2.8143x; 2.8143x over previous
import functools

import jax
import jax.numpy as jnp
from jax import lax
from jax.experimental import pallas as pl
from jax.experimental.pallas import tpu as pltpu
from jax.experimental.pallas import tpu_sc as plsc

_B, _L = 16384, 50
_H = 64
_N = _B * _L
_NC, _NS = 2, 16
_NW = _NC * _NS
_ROWS_PER_TILE = _N // _NW
_CHUNK = 128
_NCHUNK = _ROWS_PER_TILE // _CHUNK

_TBLK = 2048
_GRID = _N // _TBLK


_sc_mesh = plsc.VectorSubcoreMesh(core_axis_name="c", subcore_axis_name="s")


@functools.partial(
    pl.kernel,
    mesh=_sc_mesh,
    out_type=jax.ShapeDtypeStruct((_N, _H), jnp.float32),
    scratch_types=[
        pltpu.VMEM((_NCHUNK, _CHUNK), jnp.int32),
        pltpu.VMEM((_CHUNK, _H), jnp.float32),
        pltpu.SemaphoreType.DMA,
    ],
    compiler_params=pltpu.CompilerParams(use_tc_tiling_on_sc=False),
)
def _sc_gather(table_hbm, idx_hbm, out_hbm, idx_v, buf, sem):
    wid = lax.axis_index("s") * _NC + lax.axis_index("c")
    row0 = wid * _ROWS_PER_TILE
    pltpu.sync_copy(idx_hbm.at[pl.ds(wid * _NCHUNK, _NCHUNK)], idx_v)

    def body(j, carry):
        pltpu.async_copy(table_hbm.at[idx_v.at[j]], buf, sem).wait()
        pltpu.sync_copy(buf, out_hbm.at[pl.ds(row0 + j * _CHUNK, _CHUNK)])
        return carry

    lax.fori_loop(0, _NCHUNK, body, 0)


def _tc_body(rep_ref, pos_ref, ptab_ref, w2_ref, b2_ref, out_ref):
    x = rep_ref[...]
    p = pos_ref[...].reshape(_TBLK, 1)
    onehot = (p == lax.broadcasted_iota(jnp.int32, (_TBLK, _H), 1))
    x = x + jnp.dot(onehot.astype(jnp.float32), ptab_ref[...],
                    preferred_element_type=jnp.float32)
    mean = jnp.mean(x, axis=1, keepdims=True)
    xc = x - mean
    var = jnp.mean(xc * xc, axis=1, keepdims=True)
    xn = xc * lax.rsqrt(var + 1e-5)
    out_ref[...] = jnp.dot(xn, w2_ref[...],
                           preferred_element_type=jnp.float32) + b2_ref[...]


_tc_call = pl.pallas_call(
    _tc_body,
    grid=(_GRID,),
    in_specs=[
        pl.BlockSpec((_TBLK, _H), lambda i: (i, 0)),
        pl.BlockSpec((1, 1, _TBLK), lambda i: (i, 0, 0)),
        pl.BlockSpec((_H, _H), lambda i: (0, 0)),
        pl.BlockSpec((_H, _H), lambda i: (0, 0)),
        pl.BlockSpec((1, _H), lambda i: (0, 0)),
    ],
    out_specs=pl.BlockSpec((_TBLK, _H), lambda i: (i, 0)),
    out_shape=jax.ShapeDtypeStruct((_N, _H), jnp.float32),
    compiler_params=pltpu.CompilerParams(
        dimension_semantics=("arbitrary",)),
)


def kernel(input, pos, token_table, pos_table, gamma, beta, W, b):
    idx = input.reshape(_N // _CHUNK, _CHUNK)
    rep = _sc_gather(token_table, idx)
    posr = pos.reshape(_GRID, 1, _TBLK)
    ptab = jnp.zeros((_H, _H), jnp.float32).at[:pos_table.shape[0]].set(pos_table)
    w2 = gamma[:, None] * W.T
    b2 = (beta @ W.T + b).reshape(1, _H)
    out = _tc_call(rep, posr, ptab, w2, b2)
    return out.reshape(_B, _L, _H)


# speedup vs baseline: 3.4749x; 1.2347x over previous
import functools

import jax
import jax.numpy as jnp
from jax import lax
from jax.experimental import pallas as pl
from jax.experimental.pallas import tpu as pltpu
from jax.experimental.pallas import tpu_sc as plsc

_B, _L = 16384, 50
_H = 64
_N = _B * _L
_NC, _NS = 2, 16
_NW = _NC * _NS
_ROWS_PER_TILE = _B // _NW
_NBUF = 4

_BB = 64
_TBLK = _BB * _L
_GRID = _B // _BB


_sc_mesh = plsc.VectorSubcoreMesh(core_axis_name="c", subcore_axis_name="s")


@functools.partial(
    pl.kernel,
    mesh=_sc_mesh,
    out_type=jax.ShapeDtypeStruct((_N, _H), jnp.float32),
    scratch_types=[
        pltpu.VMEM((_ROWS_PER_TILE, _L), jnp.int32),
        pltpu.VMEM((_NBUF, _L, _H), jnp.float32),
        pltpu.SemaphoreType.DMA,
        pltpu.SemaphoreType.DMA,
        pltpu.SemaphoreType.DMA,
        pltpu.SemaphoreType.DMA,
        pltpu.SemaphoreType.DMA,
        pltpu.SemaphoreType.DMA,
        pltpu.SemaphoreType.DMA,
        pltpu.SemaphoreType.DMA,
    ],
    compiler_params=pltpu.CompilerParams(use_tc_tiling_on_sc=False),
)
def _sc_gather(table_hbm, idx_hbm, out_hbm, idx_v, bufs, sg0, sg1, sg2, sg3,
               sw0, sw1, sw2, sw3):
    wid = lax.axis_index("s") * _NC + lax.axis_index("c")
    brow0 = wid * _ROWS_PER_TILE
    trow0 = brow0 * _L
    sgs = [sg0, sg1, sg2, sg3]
    sws = [sw0, sw1, sw2, sw3]
    pltpu.sync_copy(idx_hbm.at[pl.ds(brow0, _ROWS_PER_TILE)], idx_v)

    for b in range(_NBUF):
        pltpu.async_copy(table_hbm.at[idx_v.at[b]], bufs.at[b], sgs[b])

    def body(i, carry):
        r = i * _NBUF
        for b in range(_NBUF):
            pltpu.make_async_copy(
                table_hbm.at[idx_v.at[r + b]], bufs.at[b], sgs[b]).wait()
            pltpu.async_copy(
                bufs.at[b], out_hbm.at[pl.ds(trow0 + (r + b) * _L, _L)],
                sws[b])

            nxt = r + b + _NBUF

            @pl.when(nxt < _ROWS_PER_TILE)
            def _():
                pltpu.make_async_copy(
                    bufs.at[b],
                    out_hbm.at[pl.ds(trow0 + (r + b) * _L, _L)],
                    sws[b]).wait()
                pltpu.async_copy(table_hbm.at[idx_v.at[nxt]], bufs.at[b],
                                 sgs[b])

        return carry

    lax.fori_loop(0, _ROWS_PER_TILE // _NBUF, body, 0)
    for b in range(_NBUF):
        last = _ROWS_PER_TILE - _NBUF + b
        pltpu.make_async_copy(
            bufs.at[b], out_hbm.at[pl.ds(trow0 + last * _L, _L)],
            sws[b]).wait()


def _tc_body(rep_ref, pos_ref, ptab_ref, w2_ref, b2_ref, out_ref):
    x = rep_ref[...]
    p = pos_ref[...].reshape(_TBLK, 1)
    onehot = (p == lax.broadcasted_iota(jnp.int32, (_TBLK, _H), 1))
    x = x + jnp.dot(onehot.astype(jnp.float32), ptab_ref[...],
                    preferred_element_type=jnp.float32)
    mean = jnp.mean(x, axis=1, keepdims=True)
    xc = x - mean
    var = jnp.mean(xc * xc, axis=1, keepdims=True)
    xn = xc * lax.rsqrt(var + 1e-5)
    y = jnp.dot(xn, w2_ref[...], preferred_element_type=jnp.float32) \
        + b2_ref[...]
    out_ref[...] = y.reshape(_BB, _L, _H)


_tc_call = pl.pallas_call(
    _tc_body,
    grid=(_GRID,),
    in_specs=[
        pl.BlockSpec((_TBLK, _H), lambda i: (i, 0)),
        pl.BlockSpec((1, 1, _TBLK), lambda i: (i, 0, 0)),
        pl.BlockSpec((_H, _H), lambda i: (0, 0)),
        pl.BlockSpec((_H, _H), lambda i: (0, 0)),
        pl.BlockSpec((1, _H), lambda i: (0, 0)),
    ],
    out_specs=pl.BlockSpec((_BB, _L, _H), lambda i: (i, 0, 0)),
    out_shape=jax.ShapeDtypeStruct((_B, _L, _H), jnp.float32),
    compiler_params=pltpu.CompilerParams(
        dimension_semantics=("arbitrary",)),
)


def kernel(input, pos, token_table, pos_table, gamma, beta, W, b):
    rep = _sc_gather(token_table, input)
    posr = pos.reshape(_GRID, 1, _TBLK)
    ptab = jnp.zeros((_H, _H), jnp.float32).at[:pos_table.shape[0]].set(pos_table)
    w2 = gamma[:, None] * W.T
    b2 = (beta @ W.T + b).reshape(1, _H)
    return _tc_call(rep, posr, ptab, w2, b2)
